# Initial kernel scaffold; baseline (speedup 1.0000x reference)
import jax
import jax.numpy as jnp
from jax.experimental import pallas as pl


def kernel(edge_index, x, pos, normals, W_enc, b_enc, g_enc, be_enc, W_att, W_post, b_post, g_post, be_post):
    raise NotImplementedError("write your pallas kernel here")


# speedup vs baseline: 5.3757x; 5.3757x over previous
import functools

import jax
import jax.numpy as jnp
from jax import lax
from jax.experimental import pallas as pl
from jax.experimental.pallas import tpu as pltpu
from jax.experimental.pallas import tpu_sc as plsc

_NC = 2
_NS = 16
_NW = _NC * _NS


def _pick_block(n, cands):
    for c in cands:
        if n % c == 0:
            return c
    raise ValueError(f"no block size for {n}")


def _k1_body(nchunks, src2, dst2, gtab, xtab, gsrc, gdst, xj, sidx, didx, gs, gd, xb):
    wid = lax.axis_index("s") * _NC + lax.axis_index("c")
    iters = -(-nchunks // _NW)

    def chunk(i, _):
        cid = wid + i * _NW

        @pl.when(cid < nchunks)
        def _():
            base = cid * 256
            pltpu.sync_copy(src2.at[pl.ds(2 * cid, 2)], sidx)
            pltpu.sync_copy(dst2.at[pl.ds(2 * cid, 2)], didx)
            for h in range(2):
                pltpu.sync_copy(gtab.at[sidx.at[h]], gs.at[pl.ds(h * 128, 128)])
                pltpu.sync_copy(gtab.at[didx.at[h]], gd.at[pl.ds(h * 128, 128)])
                pltpu.sync_copy(xtab.at[sidx.at[h]], xb.at[pl.ds(h * 128, 128)])
            pltpu.sync_copy(gs, gsrc.at[pl.ds(base, 256), :])
            pltpu.sync_copy(gd, gdst.at[pl.ds(base, 256), :])
            pltpu.sync_copy(xb, xj.at[pl.ds(base, 256), :])

    lax.fori_loop(0, iters, chunk, None)


def _run_k1(src2, dst2, gtab, xtab, e_total):
    mesh = plsc.VectorSubcoreMesh(core_axis_name="c", subcore_axis_name="s",
                                  num_cores=_NC, num_subcores=_NS)
    nchunks = e_total // 256
    f = pl.kernel(
        functools.partial(_k1_body, nchunks),
        out_type=[jax.ShapeDtypeStruct((e_total, 16), jnp.float32),
                  jax.ShapeDtypeStruct((e_total, 16), jnp.float32),
                  jax.ShapeDtypeStruct((e_total, 32), jnp.float32)],
        mesh=mesh,
        scratch_types=[
            pltpu.VMEM((2, 128), jnp.int32),
            pltpu.VMEM((2, 128), jnp.int32),
            pltpu.VMEM((256, 16), jnp.float32),
            pltpu.VMEM((256, 16), jnp.float32),
            pltpu.VMEM((256, 32), jnp.float32),
        ],
        compiler_params=pltpu.CompilerParams(use_tc_tiling_on_sc=False),
    )
    return f(src2, dst2, gtab, xtab)


def _rel_t(gs, gd):
    eye = jax.lax.broadcasted_iota(jnp.int32, (16, 16), 0) == \
        jax.lax.broadcasted_iota(jnp.int32, (16, 16), 1)
    eye = eye.astype(jnp.float32)
    gst = lax.dot_general(eye, gs, (((1,), (1,)), ((), ())))
    gdt = lax.dot_general(eye, gd, (((1,), (1,)), ((), ())))
    d0 = gst[0:1] - gdt[0:1]
    d1 = gst[1:2] - gdt[1:2]
    d2 = gst[2:3] - gdt[2:3]
    a0, a1, a2 = gdt[3:4], gdt[4:5], gdt[5:6]
    b0, b1, b2 = gst[3:4], gst[4:5], gst[5:6]
    dist = jnp.sqrt(d0 * d0 + d1 * d1 + d2 * d2)
    li = jnp.sqrt(a0 * a0 + a1 * a1 + a2 * a2)
    lj = jnp.sqrt(b0 * b0 + b1 * b1 + b2 * b2)
    f0 = dist
    f1 = (d0 * a0 + d1 * a1 + d2 * a2) / (dist * li + 1e-10)
    f2 = (d0 * b0 + d1 * b1 + d2 * b2) / (dist * lj + 1e-10)
    f3 = (a0 * b0 + a1 * b1 + a2 * b2) / (li * lj + 1e-10)
    uq0, uq1, uq2 = d1 * a2 - d2 * a1, d2 * a0 - d0 * a2, d0 * a1 - d1 * a0
    vq0, vq1, vq2 = uq1 * a2 - uq2 * a1, uq2 * a0 - uq0 * a2, uq0 * a1 - uq1 * a0
    uk0, uk1, uk2 = d1 * b2 - d2 * b1, d2 * b0 - d0 * b2, d0 * b1 - d1 * b0
    vk0, vk1, vk2 = uk1 * b2 - uk2 * b1, uk2 * b0 - uk0 * b2, uk0 * b1 - uk1 * b0
    luq = jnp.sqrt(uq0 * uq0 + uq1 * uq1 + uq2 * uq2)
    lvq = jnp.sqrt(vq0 * vq0 + vq1 * vq1 + vq2 * vq2)
    luk = jnp.sqrt(uk0 * uk0 + uk1 * uk1 + uk2 * uk2)
    lvk = jnp.sqrt(vk0 * vk0 + vk1 * vk1 + vk2 * vk2)
    f4 = (uq0 * uk0 + uq1 * uk1 + uq2 * uk2) / (luq * luk + 1e-10)
    f5 = (vq0 * vk0 + vq1 * vk1 + vq2 * vk2) / (lvq * lvk + 1e-10)
    f6 = (uq0 * vk0 + uq1 * vk1 + uq2 * vk2) / (luq * lvk + 1e-10)
    f7 = (vq0 * uk0 + vq1 * uk1 + vq2 * uk2) / (lvq * luk + 1e-10)
    return jnp.concatenate([f0, f1, f2, f3, f4, f5, f6, f7], axis=0)


def _k2_body(gs_ref, gd_ref, wenc_ref, benc_ref, out_ref):
    relT = _rel_t(gs_ref[...], gd_ref[...])
    hT = lax.dot_general(wenc_ref[...], relT, (((0,), (0,)), ((), ())))
    hT = hT + benc_ref[...]

    @pl.when(pl.program_id(0) == 0)
    def _():
        out_ref[...] = jnp.zeros_like(out_ref)

    out_ref[:, 0:1] += jnp.sum(hT, axis=1, keepdims=True)
    out_ref[:, 1:2] += jnp.sum(hT * hT, axis=1, keepdims=True)


def _run_k2(gsrc, gdst, w_enc, b_enc, e_total, be):
    return pl.pallas_call(
        _k2_body,
        grid=(e_total // be,),
        in_specs=[pl.BlockSpec((be, 16), lambda i: (i, 0)),
                  pl.BlockSpec((be, 16), lambda i: (i, 0)),
                  pl.BlockSpec((8, 32), lambda i: (0, 0)),
                  pl.BlockSpec((32, 1), lambda i: (0, 0))],
        out_specs=pl.BlockSpec((32, 2), lambda i: (0, 0)),
        out_shape=jax.ShapeDtypeStruct((32, 2), jnp.float32),
    )(gsrc, gdst, w_enc, b_enc)


def _k3_body(gs_ref, gd_ref, xj_ref, wenc_ref, scale_ref, shift_ref, watt_ref, v_ref):
    relT = _rel_t(gs_ref[...], gd_ref[...])
    h = lax.dot_general(relT, wenc_ref[...], (((0,), (0,)), ((), ())))
    enc = h * scale_ref[...] + shift_ref[...]
    enc = jnp.where(enc >= 0, enc, 0.2 * enc)
    local = jnp.concatenate([xj_ref[...], enc], axis=1)
    att = jnp.dot(local, watt_ref[...])
    att = jnp.minimum(jnp.maximum(att, -60.0), 60.0)
    e = jnp.exp(att)
    w = e * local
    v_ref[...] = jnp.concatenate(
        [e[:, 0:16], w[:, 0:16], e[:, 16:32], w[:, 16:32],
         e[:, 32:48], w[:, 32:48], e[:, 48:64], w[:, 48:64]], axis=1)


def _run_k3(gsrc, gdst, xj, w_enc, scale, shift, w_att, e_total, be):
    return pl.pallas_call(
        _k3_body,
        grid=(e_total // be,),
        in_specs=[pl.BlockSpec((be, 16), lambda i: (i, 0)),
                  pl.BlockSpec((be, 16), lambda i: (i, 0)),
                  pl.BlockSpec((be, 32), lambda i: (i, 0)),
                  pl.BlockSpec((8, 32), lambda i: (0, 0)),
                  pl.BlockSpec((1, 32), lambda i: (0, 0)),
                  pl.BlockSpec((1, 32), lambda i: (0, 0)),
                  pl.BlockSpec((64, 64), lambda i: (0, 0))],
        out_specs=pl.BlockSpec((be, 128), lambda i: (i, 0)),
        out_shape=jax.ShapeDtypeStruct((e_total, 128), jnp.float32),
    )(gsrc, gdst, xj, w_enc, scale, shift, w_att)


def _k4_body(nchunks, n_nodes, dst2, v, zeros, acc_out, didx, vbuf, acc):
    c = lax.axis_index("c")
    s = lax.axis_index("s")
    rows = n_nodes // _NS
    iters = -(-nchunks // _NS)

    for p in range(2):
        q = c + 2 * p
        pltpu.sync_copy(zeros, acc.at[pl.ds(s * rows, rows)])
        plsc.subcore_barrier()

        def chunk(i, _):
            cid = s + i * _NS

            @pl.when(cid < nchunks)
            def _():
                pltpu.sync_copy(dst2.at[pl.ds(cid, 1)], didx)
                pltpu.sync_copy(v.at[pl.ds(cid * 128, 128), pl.ds(q * 32, 32)],
                                vbuf)
                pltpu.sync_copy(vbuf, acc.at[didx.at[0]], add=True)

        lax.fori_loop(0, iters, chunk, None)
        plsc.subcore_barrier()
        pltpu.sync_copy(acc.at[pl.ds(s * rows, rows)],
                        acc_out.at[q, pl.ds(s * rows, rows)])
        plsc.subcore_barrier()


def _run_k4(dst2, v, n_nodes, e_total):
    mesh = plsc.VectorSubcoreMesh(core_axis_name="c", subcore_axis_name="s",
                                  num_cores=_NC, num_subcores=_NS)
    nchunks = e_total // 128
    rows = n_nodes // _NS
    zeros = jnp.zeros((rows, 32), jnp.float32)
    f = pl.kernel(
        functools.partial(_k4_body, nchunks, n_nodes),
        out_type=jax.ShapeDtypeStruct((4, n_nodes, 32), jnp.float32),
        mesh=mesh,
        scratch_types=[
            pltpu.VMEM((1, 128), jnp.int32),
            pltpu.VMEM((128, 32), jnp.float32),
            pltpu.VMEM_SHARED((n_nodes, 32), jnp.float32),
        ],
        compiler_params=pltpu.CompilerParams(use_tc_tiling_on_sc=False),
    )
    return f(dst2, v, zeros)


def _k5a_body(acc_ref, wpost_ref, bpost_ref, z_ref, st_ref):
    a = acc_ref[...]
    agg = jnp.concatenate(
        [a[qq, :, 16:32] / (a[qq, :, 0:16] + 1e-16) for qq in range(4)], axis=1)
    z = jnp.dot(agg, wpost_ref[...]) + bpost_ref[...]
    z_ref[...] = z

    @pl.when(pl.program_id(0) == 0)
    def _():
        st_ref[...] = jnp.zeros_like(st_ref)

    st_ref[0:1, :] += jnp.sum(z, axis=0, keepdims=True)
    st_ref[1:2, :] += jnp.sum(z * z, axis=0, keepdims=True)


def _run_k5a(acc, w_post, b_post, n_nodes, bn):
    return pl.pallas_call(
        _k5a_body,
        grid=(n_nodes // bn,),
        in_specs=[pl.BlockSpec((4, bn, 32), lambda i: (0, i, 0)),
                  pl.BlockSpec((64, 64), lambda i: (0, 0)),
                  pl.BlockSpec((1, 64), lambda i: (0, 0))],
        out_specs=[pl.BlockSpec((bn, 64), lambda i: (i, 0)),
                   pl.BlockSpec((2, 64), lambda i: (0, 0))],
        out_shape=[jax.ShapeDtypeStruct((n_nodes, 64), jnp.float32),
                   jax.ShapeDtypeStruct((2, 64), jnp.float32)],
    )(acc, w_post, b_post)


def _k5b_body(z_ref, scale_ref, shift_ref, out_ref):
    y = z_ref[...] * scale_ref[...] + shift_ref[...]
    out_ref[...] = jnp.where(y >= 0, y, 0.2 * y)


def _run_k5b(z, scale, shift, n_nodes, bn):
    return pl.pallas_call(
        _k5b_body,
        grid=(n_nodes // bn,),
        in_specs=[pl.BlockSpec((bn, 64), lambda i: (i, 0)),
                  pl.BlockSpec((1, 64), lambda i: (0, 0)),
                  pl.BlockSpec((1, 64), lambda i: (0, 0))],
        out_specs=pl.BlockSpec((bn, 64), lambda i: (i, 0)),
        out_shape=jax.ShapeDtypeStruct((n_nodes, 64), jnp.float32),
    )(z, scale, shift)


def kernel(edge_index, x, pos, normals, W_enc, b_enc, g_enc, be_enc, W_att,
           W_post, b_post, g_post, be_post):
    n = x.shape[0]
    e = edge_index.shape[1]
    assert e % 256 == 0 and n % _NS == 0
    be = _pick_block(e, [640, 512, 256, 128, 64, 32, 16, 8])
    bn = _pick_block(n, [1000, 512, 500, 256, 200, 128, 64, 32, 16, 8])

    src2 = edge_index[0].reshape(e // 128, 128).astype(jnp.int32)
    dst2 = edge_index[1].reshape(e // 128, 128).astype(jnp.int32)
    gtab = jnp.concatenate(
        [pos, normals, jnp.zeros((n, 10), jnp.float32)], axis=1)

    gsrc, gdst, xj = _run_k1(src2, dst2, gtab, x, e)

    stats = _run_k2(gsrc, gdst, W_enc, b_enc.reshape(32, 1), e, be)
    m = stats[:, 0] / e
    var = stats[:, 1] / e - m * m
    inv = g_enc / jnp.sqrt(var + 1e-6)
    scale = inv.reshape(1, 32)
    shift = (be_enc - m * inv).reshape(1, 32)

    v = _run_k3(gsrc, gdst, xj, W_enc, scale, shift, W_att, e, be)

    acc = _run_k4(dst2, v, n, e)

    z, zst = _run_k5a(acc, W_post, b_post.reshape(1, 64), n, bn)
    m2 = zst[0] / n
    var2 = zst[1] / n - m2 * m2
    inv2 = g_post / jnp.sqrt(var2 + 1e-6)
    scale2 = inv2.reshape(1, 64)
    shift2 = (be_post - m2 * inv2).reshape(1, 64)

    return _run_k5b(z, scale2, shift2, n, bn)


# speedup vs baseline: 6.6648x; 1.2398x over previous
import functools

import jax
import jax.numpy as jnp
from jax import lax
from jax.experimental import pallas as pl
from jax.experimental.pallas import tpu as pltpu
from jax.experimental.pallas import tpu_sc as plsc

_NC = 2
_NS = 16
_NW = _NC * _NS


def _pick_block(n, cands):
    for c in cands:
        if n % c == 0:
            return c
    raise ValueError(f"no block size for {n}")


def _k1_body(nchunks, src2, dst2, gtab, xtab, gsrc, gdst, xj, sidx, didx, gs, gd, xb,
             isem, gsem, wsem):
    wid = lax.axis_index("s") * _NC + lax.axis_index("c")
    iters = -(-nchunks // _NW)

    def chunk(i, _):
        cid = wid + i * _NW

        @pl.when(cid < nchunks)
        def _():
            base = cid * 256
            c1 = pltpu.async_copy(src2.at[pl.ds(2 * cid, 2)], sidx, isem)
            c2 = pltpu.async_copy(dst2.at[pl.ds(2 * cid, 2)], didx, isem)
            c1.wait()
            c2.wait()
            ds = []
            for h in range(2):
                ds.append(pltpu.async_copy(
                    gtab.at[sidx.at[h]], gs.at[pl.ds(h * 128, 128)], gsem))
                ds.append(pltpu.async_copy(
                    gtab.at[didx.at[h]], gd.at[pl.ds(h * 128, 128)], gsem))
                ds.append(pltpu.async_copy(
                    xtab.at[sidx.at[h]], xb.at[pl.ds(h * 128, 128)], gsem))
            for d in ds:
                d.wait()
            w1 = pltpu.async_copy(gs, gsrc.at[pl.ds(base, 256), :], wsem)
            w2 = pltpu.async_copy(gd, gdst.at[pl.ds(base, 256), :], wsem)
            w3 = pltpu.async_copy(xb, xj.at[pl.ds(base, 256), :], wsem)
            w1.wait()
            w2.wait()
            w3.wait()

    lax.fori_loop(0, iters, chunk, None)


def _run_k1(src2, dst2, gtab, xtab, e_total):
    mesh = plsc.VectorSubcoreMesh(core_axis_name="c", subcore_axis_name="s",
                                  num_cores=_NC, num_subcores=_NS)
    nchunks = e_total // 256
    f = pl.kernel(
        functools.partial(_k1_body, nchunks),
        out_type=[jax.ShapeDtypeStruct((e_total, 16), jnp.float32),
                  jax.ShapeDtypeStruct((e_total, 16), jnp.float32),
                  jax.ShapeDtypeStruct((e_total, 32), jnp.float32)],
        mesh=mesh,
        scratch_types=[
            pltpu.VMEM((2, 128), jnp.int32),
            pltpu.VMEM((2, 128), jnp.int32),
            pltpu.VMEM((256, 16), jnp.float32),
            pltpu.VMEM((256, 16), jnp.float32),
            pltpu.VMEM((256, 32), jnp.float32),
            pltpu.SemaphoreType.DMA,
            pltpu.SemaphoreType.DMA,
            pltpu.SemaphoreType.DMA,
        ],
        compiler_params=pltpu.CompilerParams(use_tc_tiling_on_sc=False),
    )
    return f(src2, dst2, gtab, xtab)


def _rel_t(gs, gd):
    eye = jax.lax.broadcasted_iota(jnp.int32, (16, 16), 0) == \
        jax.lax.broadcasted_iota(jnp.int32, (16, 16), 1)
    eye = eye.astype(jnp.float32)
    gst = lax.dot_general(eye, gs, (((1,), (1,)), ((), ())))
    gdt = lax.dot_general(eye, gd, (((1,), (1,)), ((), ())))
    d0 = gst[0:1] - gdt[0:1]
    d1 = gst[1:2] - gdt[1:2]
    d2 = gst[2:3] - gdt[2:3]
    a0, a1, a2 = gdt[3:4], gdt[4:5], gdt[5:6]
    b0, b1, b2 = gst[3:4], gst[4:5], gst[5:6]
    dist = jnp.sqrt(d0 * d0 + d1 * d1 + d2 * d2)
    li = jnp.sqrt(a0 * a0 + a1 * a1 + a2 * a2)
    lj = jnp.sqrt(b0 * b0 + b1 * b1 + b2 * b2)
    f0 = dist
    f1 = (d0 * a0 + d1 * a1 + d2 * a2) / (dist * li + 1e-10)
    f2 = (d0 * b0 + d1 * b1 + d2 * b2) / (dist * lj + 1e-10)
    f3 = (a0 * b0 + a1 * b1 + a2 * b2) / (li * lj + 1e-10)
    uq0, uq1, uq2 = d1 * a2 - d2 * a1, d2 * a0 - d0 * a2, d0 * a1 - d1 * a0
    vq0, vq1, vq2 = uq1 * a2 - uq2 * a1, uq2 * a0 - uq0 * a2, uq0 * a1 - uq1 * a0
    uk0, uk1, uk2 = d1 * b2 - d2 * b1, d2 * b0 - d0 * b2, d0 * b1 - d1 * b0
    vk0, vk1, vk2 = uk1 * b2 - uk2 * b1, uk2 * b0 - uk0 * b2, uk0 * b1 - uk1 * b0
    luq = jnp.sqrt(uq0 * uq0 + uq1 * uq1 + uq2 * uq2)
    lvq = jnp.sqrt(vq0 * vq0 + vq1 * vq1 + vq2 * vq2)
    luk = jnp.sqrt(uk0 * uk0 + uk1 * uk1 + uk2 * uk2)
    lvk = jnp.sqrt(vk0 * vk0 + vk1 * vk1 + vk2 * vk2)
    f4 = (uq0 * uk0 + uq1 * uk1 + uq2 * uk2) / (luq * luk + 1e-10)
    f5 = (vq0 * vk0 + vq1 * vk1 + vq2 * vk2) / (lvq * lvk + 1e-10)
    f6 = (uq0 * vk0 + uq1 * vk1 + uq2 * vk2) / (luq * lvk + 1e-10)
    f7 = (vq0 * uk0 + vq1 * uk1 + vq2 * uk2) / (lvq * luk + 1e-10)
    return jnp.concatenate([f0, f1, f2, f3, f4, f5, f6, f7], axis=0)


def _k2_body(gs_ref, gd_ref, wenc_ref, benc_ref, out_ref):
    relT = _rel_t(gs_ref[...], gd_ref[...])
    hT = lax.dot_general(wenc_ref[...], relT, (((0,), (0,)), ((), ())))
    hT = hT + benc_ref[...]

    @pl.when(pl.program_id(0) == 0)
    def _():
        out_ref[...] = jnp.zeros_like(out_ref)

    out_ref[:, 0:1] += jnp.sum(hT, axis=1, keepdims=True)
    out_ref[:, 1:2] += jnp.sum(hT * hT, axis=1, keepdims=True)


def _run_k2(gsrc, gdst, w_enc, b_enc, e_total, be):
    return pl.pallas_call(
        _k2_body,
        grid=(e_total // be,),
        in_specs=[pl.BlockSpec((be, 16), lambda i: (i, 0)),
                  pl.BlockSpec((be, 16), lambda i: (i, 0)),
                  pl.BlockSpec((8, 32), lambda i: (0, 0)),
                  pl.BlockSpec((32, 1), lambda i: (0, 0))],
        out_specs=pl.BlockSpec((32, 2), lambda i: (0, 0)),
        out_shape=jax.ShapeDtypeStruct((32, 2), jnp.float32),
    )(gsrc, gdst, w_enc, b_enc)


def _k3_body(gs_ref, gd_ref, xj_ref, wenc_ref, scale_ref, shift_ref, watt_ref, v_ref):
    relT = _rel_t(gs_ref[...], gd_ref[...])
    h = lax.dot_general(relT, wenc_ref[...], (((0,), (0,)), ((), ())))
    enc = h * scale_ref[...] + shift_ref[...]
    enc = jnp.where(enc >= 0, enc, 0.2 * enc)
    local = jnp.concatenate([xj_ref[...], enc], axis=1)
    att = jnp.dot(local, watt_ref[...])
    att = jnp.minimum(jnp.maximum(att, -60.0), 60.0)
    e = jnp.exp(att)
    w = e * local
    v_ref[...] = jnp.concatenate(
        [e[:, 0:16], w[:, 0:16], e[:, 16:32], w[:, 16:32],
         e[:, 32:48], w[:, 32:48], e[:, 48:64], w[:, 48:64]], axis=1)


def _run_k3(gsrc, gdst, xj, w_enc, scale, shift, w_att, e_total, be):
    return pl.pallas_call(
        _k3_body,
        grid=(e_total // be,),
        in_specs=[pl.BlockSpec((be, 16), lambda i: (i, 0)),
                  pl.BlockSpec((be, 16), lambda i: (i, 0)),
                  pl.BlockSpec((be, 32), lambda i: (i, 0)),
                  pl.BlockSpec((8, 32), lambda i: (0, 0)),
                  pl.BlockSpec((1, 32), lambda i: (0, 0)),
                  pl.BlockSpec((1, 32), lambda i: (0, 0)),
                  pl.BlockSpec((64, 64), lambda i: (0, 0))],
        out_specs=pl.BlockSpec((be, 128), lambda i: (i, 0)),
        out_shape=jax.ShapeDtypeStruct((e_total, 128), jnp.float32),
    )(gsrc, gdst, xj, w_enc, scale, shift, w_att)


def _k4_body(nchunks, n_nodes, dst2, v, zeros, acc_out, didx, vbuf, acc, lsem):
    c = lax.axis_index("c")
    s = lax.axis_index("s")
    rows = n_nodes // _NS
    iters = -(-nchunks // _NS)
    osteps = -(-iters // 2)

    for p in range(2):
        q = c + 2 * p
        pltpu.sync_copy(zeros, acc.at[pl.ds(s * rows, rows)])
        plsc.subcore_barrier()

        def start_loads(it, b):
            cid = s + it * _NS

            @pl.when(cid < nchunks)
            def _():
                pltpu.async_copy(dst2.at[pl.ds(cid, 1)],
                                 didx.at[pl.ds(b, 1)], lsem)
                pltpu.async_copy(
                    v.at[pl.ds(cid * 128, 128), pl.ds(q * 32, 32)],
                    vbuf.at[b], lsem)

        def finish(it, b):
            cid = s + it * _NS

            @pl.when(cid < nchunks)
            def _():
                pltpu.make_async_copy(dst2.at[pl.ds(cid, 1)],
                                      didx.at[pl.ds(b, 1)], lsem).wait()
                pltpu.make_async_copy(
                    v.at[pl.ds(cid * 128, 128), pl.ds(q * 32, 32)],
                    vbuf.at[b], lsem).wait()
                pltpu.sync_copy(vbuf.at[b], acc.at[didx.at[b]], add=True)

        start_loads(0, 0)
        start_loads(1, 1)

        def outer(o, _):
            it = o * 2
            finish(it, 0)
            start_loads(it + 2, 0)
            finish(it + 1, 1)
            start_loads(it + 3, 1)

        lax.fori_loop(0, osteps, outer, None)
        plsc.subcore_barrier()
        pltpu.sync_copy(acc.at[pl.ds(s * rows, rows)],
                        acc_out.at[q, pl.ds(s * rows, rows)])
        plsc.subcore_barrier()


def _run_k4(dst2, v, n_nodes, e_total):
    mesh = plsc.VectorSubcoreMesh(core_axis_name="c", subcore_axis_name="s",
                                  num_cores=_NC, num_subcores=_NS)
    nchunks = e_total // 128
    rows = n_nodes // _NS
    zeros = jnp.zeros((rows, 32), jnp.float32)
    f = pl.kernel(
        functools.partial(_k4_body, nchunks, n_nodes),
        out_type=jax.ShapeDtypeStruct((4, n_nodes, 32), jnp.float32),
        mesh=mesh,
        scratch_types=[
            pltpu.VMEM((2, 128), jnp.int32),
            pltpu.VMEM((2, 128, 32), jnp.float32),
            pltpu.VMEM_SHARED((n_nodes, 32), jnp.float32),
            pltpu.SemaphoreType.DMA,
        ],
        compiler_params=pltpu.CompilerParams(use_tc_tiling_on_sc=False),
    )
    return f(dst2, v, zeros)


def _k5a_body(acc_ref, wpost_ref, bpost_ref, z_ref, st_ref):
    a = acc_ref[...]
    agg = jnp.concatenate(
        [a[qq, :, 16:32] / (a[qq, :, 0:16] + 1e-16) for qq in range(4)], axis=1)
    z = jnp.dot(agg, wpost_ref[...]) + bpost_ref[...]
    z_ref[...] = z

    @pl.when(pl.program_id(0) == 0)
    def _():
        st_ref[...] = jnp.zeros_like(st_ref)

    st_ref[0:1, :] += jnp.sum(z, axis=0, keepdims=True)
    st_ref[1:2, :] += jnp.sum(z * z, axis=0, keepdims=True)


def _run_k5a(acc, w_post, b_post, n_nodes, bn):
    return pl.pallas_call(
        _k5a_body,
        grid=(n_nodes // bn,),
        in_specs=[pl.BlockSpec((4, bn, 32), lambda i: (0, i, 0)),
                  pl.BlockSpec((64, 64), lambda i: (0, 0)),
                  pl.BlockSpec((1, 64), lambda i: (0, 0))],
        out_specs=[pl.BlockSpec((bn, 64), lambda i: (i, 0)),
                   pl.BlockSpec((2, 64), lambda i: (0, 0))],
        out_shape=[jax.ShapeDtypeStruct((n_nodes, 64), jnp.float32),
                   jax.ShapeDtypeStruct((2, 64), jnp.float32)],
    )(acc, w_post, b_post)


def _k5b_body(z_ref, scale_ref, shift_ref, out_ref):
    y = z_ref[...] * scale_ref[...] + shift_ref[...]
    out_ref[...] = jnp.where(y >= 0, y, 0.2 * y)


def _run_k5b(z, scale, shift, n_nodes, bn):
    return pl.pallas_call(
        _k5b_body,
        grid=(n_nodes // bn,),
        in_specs=[pl.BlockSpec((bn, 64), lambda i: (i, 0)),
                  pl.BlockSpec((1, 64), lambda i: (0, 0)),
                  pl.BlockSpec((1, 64), lambda i: (0, 0))],
        out_specs=pl.BlockSpec((bn, 64), lambda i: (i, 0)),
        out_shape=jax.ShapeDtypeStruct((n_nodes, 64), jnp.float32),
    )(z, scale, shift)


def kernel(edge_index, x, pos, normals, W_enc, b_enc, g_enc, be_enc, W_att,
           W_post, b_post, g_post, be_post):
    n = x.shape[0]
    e = edge_index.shape[1]
    assert e % 256 == 0 and n % _NS == 0
    be = _pick_block(e, [640, 512, 256, 128, 64, 32, 16, 8])
    bn = _pick_block(n, [1000, 512, 500, 256, 200, 128, 64, 32, 16, 8])

    src2 = edge_index[0].reshape(e // 128, 128).astype(jnp.int32)
    dst2 = edge_index[1].reshape(e // 128, 128).astype(jnp.int32)
    gtab = jnp.concatenate(
        [pos, normals, jnp.zeros((n, 10), jnp.float32)], axis=1)

    gsrc, gdst, xj = _run_k1(src2, dst2, gtab, x, e)

    stats = _run_k2(gsrc, gdst, W_enc, b_enc.reshape(32, 1), e, be)
    m = stats[:, 0] / e
    var = stats[:, 1] / e - m * m
    inv = g_enc / jnp.sqrt(var + 1e-6)
    scale = inv.reshape(1, 32)
    shift = (be_enc - m * inv).reshape(1, 32)

    v = _run_k3(gsrc, gdst, xj, W_enc, scale, shift, W_att, e, be)

    acc = _run_k4(dst2, v, n, e)

    z, zst = _run_k5a(acc, W_post, b_post.reshape(1, 64), n, bn)
    m2 = zst[0] / n
    var2 = zst[1] / n - m2 * m2
    inv2 = g_post / jnp.sqrt(var2 + 1e-6)
    scale2 = inv2.reshape(1, 64)
    shift2 = (be_post - m2 * inv2).reshape(1, 64)

    return _run_k5b(z, scale2, shift2, n, bn)


# speedup vs baseline: 6.7847x; 1.0180x over previous
import functools

import jax
import jax.numpy as jnp
from jax import lax
from jax.experimental import pallas as pl
from jax.experimental.pallas import tpu as pltpu
from jax.experimental.pallas import tpu_sc as plsc

_NC = 2
_NS = 16
_NW = _NC * _NS


def _pick_block(n, cands):
    for c in cands:
        if n % c == 0:
            return c
    raise ValueError(f"no block size for {n}")


def _k1_body(nchunks, src2, dst2, gtab, xtab, gsrc, gdst, xj, sidx, didx,
             gs, gd, xb, isem, gsem, wsem):
    wid = lax.axis_index("s") * _NC + lax.axis_index("c")
    iters = -(-nchunks // _NW)
    osteps = -(-iters // 2)

    def start_idx(it, b):
        cid = wid + it * _NW

        @pl.when(cid < nchunks)
        def _():
            pltpu.async_copy(src2.at[pl.ds(2 * cid, 2)], sidx.at[b],
                             isem.at[b])
            pltpu.async_copy(dst2.at[pl.ds(2 * cid, 2)], didx.at[b],
                             isem.at[b])

    def wait_idx(it, b):
        cid = wid + it * _NW

        @pl.when(cid < nchunks)
        def _():
            pltpu.make_async_copy(src2.at[pl.ds(2 * cid, 2)], sidx.at[b],
                                  isem.at[b]).wait()
            pltpu.make_async_copy(dst2.at[pl.ds(2 * cid, 2)], didx.at[b],
                                  isem.at[b]).wait()

    def start_gathers(it, b):
        cid = wid + it * _NW

        @pl.when(cid < nchunks)
        def _():
            for h in range(2):
                pltpu.async_copy(gtab.at[sidx.at[b, h]],
                                 gs.at[b, pl.ds(h * 128, 128)], gsem.at[b])
                pltpu.async_copy(gtab.at[didx.at[b, h]],
                                 gd.at[b, pl.ds(h * 128, 128)], gsem.at[b])
                pltpu.async_copy(xtab.at[sidx.at[b, h]],
                                 xb.at[b, pl.ds(h * 128, 128)], gsem.at[b])

    def wait_gathers(it, b):
        cid = wid + it * _NW

        @pl.when(cid < nchunks)
        def _():
            for h in range(2):
                pltpu.make_async_copy(gtab.at[sidx.at[b, h]],
                                      gs.at[b, pl.ds(h * 128, 128)],
                                      gsem.at[b]).wait()
                pltpu.make_async_copy(gtab.at[didx.at[b, h]],
                                      gd.at[b, pl.ds(h * 128, 128)],
                                      gsem.at[b]).wait()
                pltpu.make_async_copy(xtab.at[sidx.at[b, h]],
                                      xb.at[b, pl.ds(h * 128, 128)],
                                      gsem.at[b]).wait()

    def writes(it, b):
        cid = wid + it * _NW

        @pl.when(cid < nchunks)
        def _():
            base = cid * 256
            w1 = pltpu.async_copy(gs.at[b], gsrc.at[pl.ds(base, 256), :], wsem)
            w2 = pltpu.async_copy(gd.at[b], gdst.at[pl.ds(base, 256), :], wsem)
            w3 = pltpu.async_copy(xb.at[b], xj.at[pl.ds(base, 256), :], wsem)
            w1.wait()
            w2.wait()
            w3.wait()

    def step(it, b):
        wait_idx(it + 1, 1 - b)
        start_gathers(it + 1, 1 - b)
        wait_gathers(it, b)
        start_idx(it + 2, b)
        writes(it, b)

    start_idx(0, 0)
    wait_idx(0, 0)
    start_gathers(0, 0)
    start_idx(1, 1)

    def outer(o, _):
        it = o * 2
        step(it, 0)
        step(it + 1, 1)

    lax.fori_loop(0, osteps, outer, None)


def _run_k1(src2, dst2, gtab, xtab, e_total):
    mesh = plsc.VectorSubcoreMesh(core_axis_name="c", subcore_axis_name="s",
                                  num_cores=_NC, num_subcores=_NS)
    nchunks = e_total // 256
    f = pl.kernel(
        functools.partial(_k1_body, nchunks),
        out_type=[jax.ShapeDtypeStruct((e_total, 16), jnp.float32),
                  jax.ShapeDtypeStruct((e_total, 16), jnp.float32),
                  jax.ShapeDtypeStruct((e_total, 32), jnp.float32)],
        mesh=mesh,
        scratch_types=[
            pltpu.VMEM((2, 2, 128), jnp.int32),
            pltpu.VMEM((2, 2, 128), jnp.int32),
            pltpu.VMEM((2, 256, 16), jnp.float32),
            pltpu.VMEM((2, 256, 16), jnp.float32),
            pltpu.VMEM((2, 256, 32), jnp.float32),
            pltpu.SemaphoreType.DMA((2,)),
            pltpu.SemaphoreType.DMA((2,)),
            pltpu.SemaphoreType.DMA,
        ],
        compiler_params=pltpu.CompilerParams(use_tc_tiling_on_sc=False),
    )
    return f(src2, dst2, gtab, xtab)


def _rel_t(gs, gd):
    eye = jax.lax.broadcasted_iota(jnp.int32, (16, 16), 0) == \
        jax.lax.broadcasted_iota(jnp.int32, (16, 16), 1)
    eye = eye.astype(jnp.float32)
    gst = lax.dot_general(eye, gs, (((1,), (1,)), ((), ())))
    gdt = lax.dot_general(eye, gd, (((1,), (1,)), ((), ())))
    d0 = gst[0:1] - gdt[0:1]
    d1 = gst[1:2] - gdt[1:2]
    d2 = gst[2:3] - gdt[2:3]
    a0, a1, a2 = gdt[3:4], gdt[4:5], gdt[5:6]
    b0, b1, b2 = gst[3:4], gst[4:5], gst[5:6]
    dist = jnp.sqrt(d0 * d0 + d1 * d1 + d2 * d2)
    li = jnp.sqrt(a0 * a0 + a1 * a1 + a2 * a2)
    lj = jnp.sqrt(b0 * b0 + b1 * b1 + b2 * b2)
    f0 = dist
    f1 = (d0 * a0 + d1 * a1 + d2 * a2) / (dist * li + 1e-10)
    f2 = (d0 * b0 + d1 * b1 + d2 * b2) / (dist * lj + 1e-10)
    f3 = (a0 * b0 + a1 * b1 + a2 * b2) / (li * lj + 1e-10)
    uq0, uq1, uq2 = d1 * a2 - d2 * a1, d2 * a0 - d0 * a2, d0 * a1 - d1 * a0
    vq0, vq1, vq2 = uq1 * a2 - uq2 * a1, uq2 * a0 - uq0 * a2, uq0 * a1 - uq1 * a0
    uk0, uk1, uk2 = d1 * b2 - d2 * b1, d2 * b0 - d0 * b2, d0 * b1 - d1 * b0
    vk0, vk1, vk2 = uk1 * b2 - uk2 * b1, uk2 * b0 - uk0 * b2, uk0 * b1 - uk1 * b0
    luq = jnp.sqrt(uq0 * uq0 + uq1 * uq1 + uq2 * uq2)
    lvq = jnp.sqrt(vq0 * vq0 + vq1 * vq1 + vq2 * vq2)
    luk = jnp.sqrt(uk0 * uk0 + uk1 * uk1 + uk2 * uk2)
    lvk = jnp.sqrt(vk0 * vk0 + vk1 * vk1 + vk2 * vk2)
    f4 = (uq0 * uk0 + uq1 * uk1 + uq2 * uk2) / (luq * luk + 1e-10)
    f5 = (vq0 * vk0 + vq1 * vk1 + vq2 * vk2) / (lvq * lvk + 1e-10)
    f6 = (uq0 * vk0 + uq1 * vk1 + uq2 * vk2) / (luq * lvk + 1e-10)
    f7 = (vq0 * uk0 + vq1 * uk1 + vq2 * uk2) / (lvq * luk + 1e-10)
    return jnp.concatenate([f0, f1, f2, f3, f4, f5, f6, f7], axis=0)


def _k2_body(gs_ref, gd_ref, wenc_ref, benc_ref, rel_ref, out_ref):
    relT = _rel_t(gs_ref[...], gd_ref[...])
    rel_ref[...] = relT
    hT = lax.dot_general(wenc_ref[...], relT, (((0,), (0,)), ((), ())))
    hT = hT + benc_ref[...]

    @pl.when(pl.program_id(0) == 0)
    def _():
        out_ref[...] = jnp.zeros_like(out_ref)

    out_ref[:, 0:1] += jnp.sum(hT, axis=1, keepdims=True)
    out_ref[:, 1:2] += jnp.sum(hT * hT, axis=1, keepdims=True)


def _run_k2(gsrc, gdst, w_enc, b_enc, e_total, be):
    return pl.pallas_call(
        _k2_body,
        grid=(e_total // be,),
        in_specs=[pl.BlockSpec((be, 16), lambda i: (i, 0)),
                  pl.BlockSpec((be, 16), lambda i: (i, 0)),
                  pl.BlockSpec((8, 32), lambda i: (0, 0)),
                  pl.BlockSpec((32, 1), lambda i: (0, 0))],
        out_specs=[pl.BlockSpec((8, be), lambda i: (0, i)),
                   pl.BlockSpec((32, 2), lambda i: (0, 0))],
        out_shape=[jax.ShapeDtypeStruct((8, e_total), jnp.float32),
                   jax.ShapeDtypeStruct((32, 2), jnp.float32)],
    )(gsrc, gdst, w_enc, b_enc)


def _k3_body(rel_ref, xj_ref, wenc_ref, scale_ref, shift_ref, watt_ref,
             ve_ref, vw_ref):
    h = lax.dot_general(rel_ref[...], wenc_ref[...],
                        (((0,), (0,)), ((), ())))
    enc = h * scale_ref[...] + shift_ref[...]
    enc = jnp.where(enc >= 0, enc, 0.2 * enc)
    local = jnp.concatenate([xj_ref[...], enc], axis=1)
    att = jnp.dot(local, watt_ref[...])
    att = jnp.minimum(jnp.maximum(att, -60.0), 60.0)
    e = jnp.exp(att)
    ve_ref[...] = e
    vw_ref[...] = e * local


def _run_k3(rel, xj, w_enc, scale, shift, w_att, e_total, be):
    return pl.pallas_call(
        _k3_body,
        grid=(e_total // be,),
        in_specs=[pl.BlockSpec((8, be), lambda i: (0, i)),
                  pl.BlockSpec((be, 32), lambda i: (i, 0)),
                  pl.BlockSpec((8, 32), lambda i: (0, 0)),
                  pl.BlockSpec((1, 32), lambda i: (0, 0)),
                  pl.BlockSpec((1, 32), lambda i: (0, 0)),
                  pl.BlockSpec((64, 64), lambda i: (0, 0))],
        out_specs=[pl.BlockSpec((be, 64), lambda i: (i, 0)),
                   pl.BlockSpec((be, 64), lambda i: (i, 0))],
        out_shape=[jax.ShapeDtypeStruct((e_total, 64), jnp.float32),
                   jax.ShapeDtypeStruct((e_total, 64), jnp.float32)],
    )(rel, xj, w_enc, scale, shift, w_att)


def _k4_body(nchunks, n_nodes, dst2, ve, vw, zeros, acc_out, didx, vbuf, acc, lsem):
    c = lax.axis_index("c")
    s = lax.axis_index("s")
    rows = n_nodes // _NS
    iters = -(-nchunks // _NS)
    osteps = -(-iters // 2)

    for p in range(2):
        q = c + 2 * p
        pltpu.sync_copy(zeros, acc.at[pl.ds(s * rows, rows)])
        plsc.subcore_barrier()

        def start_loads(it, b):
            cid = s + it * _NS

            @pl.when(cid < nchunks)
            def _():
                pltpu.async_copy(dst2.at[pl.ds(cid, 1)],
                                 didx.at[pl.ds(b, 1)], lsem.at[b])
                pltpu.async_copy(
                    ve.at[pl.ds(cid * 128, 128), pl.ds(q * 16, 16)],
                    vbuf.at[b, :, pl.ds(0, 16)], lsem.at[b])
                pltpu.async_copy(
                    vw.at[pl.ds(cid * 128, 128), pl.ds(q * 16, 16)],
                    vbuf.at[b, :, pl.ds(16, 16)], lsem.at[b])

        def finish(it, b):
            cid = s + it * _NS

            @pl.when(cid < nchunks)
            def _():
                pltpu.make_async_copy(dst2.at[pl.ds(cid, 1)],
                                      didx.at[pl.ds(b, 1)], lsem.at[b]).wait()
                pltpu.make_async_copy(
                    ve.at[pl.ds(cid * 128, 128), pl.ds(q * 16, 16)],
                    vbuf.at[b, :, pl.ds(0, 16)], lsem.at[b]).wait()
                pltpu.make_async_copy(
                    vw.at[pl.ds(cid * 128, 128), pl.ds(q * 16, 16)],
                    vbuf.at[b, :, pl.ds(16, 16)], lsem.at[b]).wait()
                pltpu.sync_copy(vbuf.at[b], acc.at[didx.at[b]], add=True)

        start_loads(0, 0)
        start_loads(1, 1)

        def outer(o, _):
            it = o * 2
            finish(it, 0)
            start_loads(it + 2, 0)
            finish(it + 1, 1)
            start_loads(it + 3, 1)

        lax.fori_loop(0, osteps, outer, None)
        plsc.subcore_barrier()
        pltpu.sync_copy(acc.at[pl.ds(s * rows, rows)],
                        acc_out.at[q, pl.ds(s * rows, rows)])
        plsc.subcore_barrier()


def _run_k4(dst2, ve, vw, n_nodes, e_total):
    mesh = plsc.VectorSubcoreMesh(core_axis_name="c", subcore_axis_name="s",
                                  num_cores=_NC, num_subcores=_NS)
    nchunks = e_total // 128
    rows = n_nodes // _NS
    zeros = jnp.zeros((rows, 32), jnp.float32)
    f = pl.kernel(
        functools.partial(_k4_body, nchunks, n_nodes),
        out_type=jax.ShapeDtypeStruct((4, n_nodes, 32), jnp.float32),
        mesh=mesh,
        scratch_types=[
            pltpu.VMEM((2, 128), jnp.int32),
            pltpu.VMEM((2, 128, 32), jnp.float32),
            pltpu.VMEM_SHARED((n_nodes, 32), jnp.float32),
            pltpu.SemaphoreType.DMA((2,)),
        ],
        compiler_params=pltpu.CompilerParams(use_tc_tiling_on_sc=False),
    )
    return f(dst2, ve, vw, zeros)


def _k5a_body(acc_ref, wpost_ref, bpost_ref, z_ref, st_ref):
    a = acc_ref[...]
    agg = jnp.concatenate(
        [a[qq, :, 16:32] / (a[qq, :, 0:16] + 1e-16) for qq in range(4)], axis=1)
    z = jnp.dot(agg, wpost_ref[...]) + bpost_ref[...]
    z_ref[...] = z

    @pl.when(pl.program_id(0) == 0)
    def _():
        st_ref[...] = jnp.zeros_like(st_ref)

    st_ref[0:1, :] += jnp.sum(z, axis=0, keepdims=True)
    st_ref[1:2, :] += jnp.sum(z * z, axis=0, keepdims=True)


def _run_k5a(acc, w_post, b_post, n_nodes, bn):
    return pl.pallas_call(
        _k5a_body,
        grid=(n_nodes // bn,),
        in_specs=[pl.BlockSpec((4, bn, 32), lambda i: (0, i, 0)),
                  pl.BlockSpec((64, 64), lambda i: (0, 0)),
                  pl.BlockSpec((1, 64), lambda i: (0, 0))],
        out_specs=[pl.BlockSpec((bn, 64), lambda i: (i, 0)),
                   pl.BlockSpec((2, 64), lambda i: (0, 0))],
        out_shape=[jax.ShapeDtypeStruct((n_nodes, 64), jnp.float32),
                   jax.ShapeDtypeStruct((2, 64), jnp.float32)],
    )(acc, w_post, b_post)


def _k5b_body(z_ref, scale_ref, shift_ref, out_ref):
    y = z_ref[...] * scale_ref[...] + shift_ref[...]
    out_ref[...] = jnp.where(y >= 0, y, 0.2 * y)


def _run_k5b(z, scale, shift, n_nodes, bn):
    return pl.pallas_call(
        _k5b_body,
        grid=(n_nodes // bn,),
        in_specs=[pl.BlockSpec((bn, 64), lambda i: (i, 0)),
                  pl.BlockSpec((1, 64), lambda i: (0, 0)),
                  pl.BlockSpec((1, 64), lambda i: (0, 0))],
        out_specs=pl.BlockSpec((bn, 64), lambda i: (i, 0)),
        out_shape=jax.ShapeDtypeStruct((n_nodes, 64), jnp.float32),
    )(z, scale, shift)


def kernel(edge_index, x, pos, normals, W_enc, b_enc, g_enc, be_enc, W_att,
           W_post, b_post, g_post, be_post):
    n = x.shape[0]
    e = edge_index.shape[1]
    assert e % 256 == 0 and n % _NS == 0
    be = _pick_block(e, [1280, 640, 512, 256, 128, 64, 32, 16, 8])
    bn = _pick_block(n, [1000, 512, 500, 256, 200, 128, 64, 32, 16, 8])

    src2 = edge_index[0].reshape(e // 128, 128).astype(jnp.int32)
    dst2 = edge_index[1].reshape(e // 128, 128).astype(jnp.int32)
    gtab = jnp.concatenate(
        [pos, normals, jnp.zeros((n, 10), jnp.float32)], axis=1)

    gsrc, gdst, xj = _run_k1(src2, dst2, gtab, x, e)

    rel, stats = _run_k2(gsrc, gdst, W_enc, b_enc.reshape(32, 1), e, be)
    m = stats[:, 0] / e
    var = stats[:, 1] / e - m * m
    inv = g_enc / jnp.sqrt(var + 1e-6)
    scale = inv.reshape(1, 32)
    shift = (be_enc - m * inv).reshape(1, 32)

    ve, vw = _run_k3(rel, xj, W_enc, scale, shift, W_att, e, be)

    acc = _run_k4(dst2, ve, vw, n, e)

    z, zst = _run_k5a(acc, W_post, b_post.reshape(1, 64), n, bn)
    m2 = zst[0] / n
    var2 = zst[1] / n - m2 * m2
    inv2 = g_post / jnp.sqrt(var2 + 1e-6)
    scale2 = inv2.reshape(1, 64)
    shift2 = (be_post - m2 * inv2).reshape(1, 64)

    return _run_k5b(z, scale2, shift2, n, bn)


# speedup vs baseline: 10.0041x; 1.4745x over previous
import functools

import jax
import jax.numpy as jnp
from jax import lax
from jax.experimental import pallas as pl
from jax.experimental.pallas import tpu as pltpu
from jax.experimental.pallas import tpu_sc as plsc

_NC = 2
_NS = 16
_NW = _NC * _NS


def _pick_block(n, cands):
    for c in cands:
        if n % c == 0:
            return c
    raise ValueError(f"no block size for {n}")


def _k1_body(nchunks, src2, dst2, gtab, xtab, gsrc, gdst, xj, sidx, didx,
             gs, gd, xb, isem, gsem, wsem):
    wid = lax.axis_index("s") * _NC + lax.axis_index("c")
    iters = -(-nchunks // _NW)
    osteps = -(-iters // 2)

    def start_idx(it, b):
        cid = wid + it * _NW

        @pl.when(cid < nchunks)
        def _():
            pltpu.async_copy(src2.at[pl.ds(2 * cid, 2)], sidx.at[b],
                             isem.at[b])
            pltpu.async_copy(dst2.at[pl.ds(2 * cid, 2)], didx.at[b],
                             isem.at[b])

    def wait_idx(it, b):
        cid = wid + it * _NW

        @pl.when(cid < nchunks)
        def _():
            pltpu.make_async_copy(src2.at[pl.ds(2 * cid, 2)], sidx.at[b],
                                  isem.at[b]).wait()
            pltpu.make_async_copy(dst2.at[pl.ds(2 * cid, 2)], didx.at[b],
                                  isem.at[b]).wait()

    def start_gathers(it, b):
        cid = wid + it * _NW

        @pl.when(cid < nchunks)
        def _():
            for h in range(2):
                pltpu.async_copy(gtab.at[sidx.at[b, h]],
                                 gs.at[b, pl.ds(h * 128, 128)], gsem.at[b])
                pltpu.async_copy(gtab.at[didx.at[b, h]],
                                 gd.at[b, pl.ds(h * 128, 128)], gsem.at[b])
                pltpu.async_copy(xtab.at[sidx.at[b, h]],
                                 xb.at[b, pl.ds(h * 128, 128)], gsem.at[b])

    def wait_gathers(it, b):
        cid = wid + it * _NW

        @pl.when(cid < nchunks)
        def _():
            for h in range(2):
                pltpu.make_async_copy(gtab.at[sidx.at[b, h]],
                                      gs.at[b, pl.ds(h * 128, 128)],
                                      gsem.at[b]).wait()
                pltpu.make_async_copy(gtab.at[didx.at[b, h]],
                                      gd.at[b, pl.ds(h * 128, 128)],
                                      gsem.at[b]).wait()
                pltpu.make_async_copy(xtab.at[sidx.at[b, h]],
                                      xb.at[b, pl.ds(h * 128, 128)],
                                      gsem.at[b]).wait()

    def writes(it, b):
        cid = wid + it * _NW

        @pl.when(cid < nchunks)
        def _():
            base = cid * 256
            w1 = pltpu.async_copy(gs.at[b], gsrc.at[pl.ds(base, 256), :], wsem)
            w2 = pltpu.async_copy(gd.at[b], gdst.at[pl.ds(base, 256), :], wsem)
            w3 = pltpu.async_copy(xb.at[b], xj.at[pl.ds(base, 256), :], wsem)
            w1.wait()
            w2.wait()
            w3.wait()

    def step(it, b):
        wait_idx(it + 1, 1 - b)
        start_gathers(it + 1, 1 - b)
        wait_gathers(it, b)
        start_idx(it + 2, b)
        writes(it, b)

    start_idx(0, 0)
    wait_idx(0, 0)
    start_gathers(0, 0)
    start_idx(1, 1)

    def outer(o, _):
        it = o * 2
        step(it, 0)
        step(it + 1, 1)

    lax.fori_loop(0, osteps, outer, None)


def _run_k1(src2, dst2, gtab, xtab, e_total):
    mesh = plsc.VectorSubcoreMesh(core_axis_name="c", subcore_axis_name="s",
                                  num_cores=_NC, num_subcores=_NS)
    nchunks = e_total // 256
    f = pl.kernel(
        functools.partial(_k1_body, nchunks),
        out_type=[jax.ShapeDtypeStruct((e_total, 16), jnp.float32),
                  jax.ShapeDtypeStruct((e_total, 16), jnp.float32),
                  jax.ShapeDtypeStruct((e_total, 32), jnp.float32)],
        mesh=mesh,
        scratch_types=[
            pltpu.VMEM((2, 2, 128), jnp.int32),
            pltpu.VMEM((2, 2, 128), jnp.int32),
            pltpu.VMEM((2, 256, 16), jnp.float32),
            pltpu.VMEM((2, 256, 16), jnp.float32),
            pltpu.VMEM((2, 256, 32), jnp.float32),
            pltpu.SemaphoreType.DMA((2,)),
            pltpu.SemaphoreType.DMA((2,)),
            pltpu.SemaphoreType.DMA,
        ],
        compiler_params=pltpu.CompilerParams(use_tc_tiling_on_sc=False),
    )
    return f(src2, dst2, gtab, xtab)


def _rel_t(gs, gd):
    eye = jax.lax.broadcasted_iota(jnp.int32, (16, 16), 0) == \
        jax.lax.broadcasted_iota(jnp.int32, (16, 16), 1)
    eye = eye.astype(jnp.float32)
    gst = lax.dot_general(eye, gs, (((1,), (1,)), ((), ())))
    gdt = lax.dot_general(eye, gd, (((1,), (1,)), ((), ())))
    d0 = gst[0:1] - gdt[0:1]
    d1 = gst[1:2] - gdt[1:2]
    d2 = gst[2:3] - gdt[2:3]
    a0, a1, a2 = gdt[3:4], gdt[4:5], gdt[5:6]
    b0, b1, b2 = gst[3:4], gst[4:5], gst[5:6]
    dist = jnp.sqrt(d0 * d0 + d1 * d1 + d2 * d2)
    li = jnp.sqrt(a0 * a0 + a1 * a1 + a2 * a2)
    lj = jnp.sqrt(b0 * b0 + b1 * b1 + b2 * b2)
    f0 = dist
    f1 = (d0 * a0 + d1 * a1 + d2 * a2) / (dist * li + 1e-10)
    f2 = (d0 * b0 + d1 * b1 + d2 * b2) / (dist * lj + 1e-10)
    f3 = (a0 * b0 + a1 * b1 + a2 * b2) / (li * lj + 1e-10)
    uq0, uq1, uq2 = d1 * a2 - d2 * a1, d2 * a0 - d0 * a2, d0 * a1 - d1 * a0
    vq0, vq1, vq2 = uq1 * a2 - uq2 * a1, uq2 * a0 - uq0 * a2, uq0 * a1 - uq1 * a0
    uk0, uk1, uk2 = d1 * b2 - d2 * b1, d2 * b0 - d0 * b2, d0 * b1 - d1 * b0
    vk0, vk1, vk2 = uk1 * b2 - uk2 * b1, uk2 * b0 - uk0 * b2, uk0 * b1 - uk1 * b0
    luq = jnp.sqrt(uq0 * uq0 + uq1 * uq1 + uq2 * uq2)
    lvq = jnp.sqrt(vq0 * vq0 + vq1 * vq1 + vq2 * vq2)
    luk = jnp.sqrt(uk0 * uk0 + uk1 * uk1 + uk2 * uk2)
    lvk = jnp.sqrt(vk0 * vk0 + vk1 * vk1 + vk2 * vk2)
    f4 = (uq0 * uk0 + uq1 * uk1 + uq2 * uk2) / (luq * luk + 1e-10)
    f5 = (vq0 * vk0 + vq1 * vk1 + vq2 * vk2) / (lvq * lvk + 1e-10)
    f6 = (uq0 * vk0 + uq1 * vk1 + uq2 * vk2) / (luq * lvk + 1e-10)
    f7 = (vq0 * uk0 + vq1 * uk1 + vq2 * uk2) / (lvq * luk + 1e-10)
    return jnp.concatenate([f0, f1, f2, f3, f4, f5, f6, f7], axis=0)


def _k2_body(gs_ref, gd_ref, wenc_ref, benc_ref, rel_ref, out_ref):
    relT = _rel_t(gs_ref[...], gd_ref[...])
    rel_ref[...] = relT
    hT = lax.dot_general(wenc_ref[...], relT, (((0,), (0,)), ((), ())))
    hT = hT + benc_ref[...]

    @pl.when(pl.program_id(0) == 0)
    def _():
        out_ref[...] = jnp.zeros_like(out_ref)

    out_ref[:, 0:1] += jnp.sum(hT, axis=1, keepdims=True)
    out_ref[:, 1:2] += jnp.sum(hT * hT, axis=1, keepdims=True)


def _run_k2(gsrc, gdst, w_enc, b_enc, e_total, be):
    return pl.pallas_call(
        _k2_body,
        grid=(e_total // be,),
        in_specs=[pl.BlockSpec((be, 16), lambda i: (i, 0)),
                  pl.BlockSpec((be, 16), lambda i: (i, 0)),
                  pl.BlockSpec((8, 32), lambda i: (0, 0)),
                  pl.BlockSpec((32, 1), lambda i: (0, 0))],
        out_specs=[pl.BlockSpec((8, be), lambda i: (0, i)),
                   pl.BlockSpec((32, 2), lambda i: (0, 0))],
        out_shape=[jax.ShapeDtypeStruct((8, e_total), jnp.float32),
                   jax.ShapeDtypeStruct((32, 2), jnp.float32)],
    )(gsrc, gdst, w_enc, b_enc)


def _k3_body(rel_ref, xj_ref, wenc_ref, scale_ref, shift_ref, watt_ref, v_ref):
    h = lax.dot_general(rel_ref[...], wenc_ref[...],
                        (((0,), (0,)), ((), ())))
    enc = h * scale_ref[...] + shift_ref[...]
    enc = jnp.where(enc >= 0, enc, 0.2 * enc)
    local = jnp.concatenate([xj_ref[...], enc], axis=1)
    att = jnp.dot(local, watt_ref[...])
    att = jnp.minimum(jnp.maximum(att, -60.0), 60.0)
    e = jnp.exp(att)
    w = e * local
    row = jax.lax.broadcasted_iota(jnp.int32, (64, 128), 0)
    col = jax.lax.broadcasted_iota(jnp.int32, (64, 128), 1)
    qq, rr = col // 32, col % 32
    pe = ((rr < 16) & (row == qq * 16 + rr)).astype(jnp.float32)
    pw = ((rr >= 16) & (row == qq * 16 + rr - 16)).astype(jnp.float32)
    v_ref[...] = jnp.dot(e, pe) + jnp.dot(w, pw)


def _run_k3(rel, xj, w_enc, scale, shift, w_att, e_total, be):
    return pl.pallas_call(
        _k3_body,
        grid=(e_total // be,),
        in_specs=[pl.BlockSpec((8, be), lambda i: (0, i)),
                  pl.BlockSpec((be, 32), lambda i: (i, 0)),
                  pl.BlockSpec((8, 32), lambda i: (0, 0)),
                  pl.BlockSpec((1, 32), lambda i: (0, 0)),
                  pl.BlockSpec((1, 32), lambda i: (0, 0)),
                  pl.BlockSpec((64, 64), lambda i: (0, 0))],
        out_specs=pl.BlockSpec((be, 128), lambda i: (i, 0)),
        out_shape=jax.ShapeDtypeStruct((e_total, 128), jnp.float32),
    )(rel, xj, w_enc, scale, shift, w_att)


def _k4_body(nchunks, n_nodes, dst2, v, zeros, acc_out, didx, vbuf, acc, lsem):
    c = lax.axis_index("c")
    s = lax.axis_index("s")
    rows = n_nodes // _NS
    iters = -(-nchunks // _NS)
    osteps = -(-iters // 2)

    for p in range(2):
        q = c + 2 * p
        pltpu.sync_copy(zeros, acc.at[pl.ds(s * rows, rows)])
        plsc.subcore_barrier()

        def start_loads(it, b):
            cid = s + it * _NS

            @pl.when(cid < nchunks)
            def _():
                pltpu.async_copy(dst2.at[pl.ds(cid, 1)],
                                 didx.at[pl.ds(b, 1)], lsem.at[b])
                pltpu.async_copy(
                    v.at[pl.ds(cid * 128, 128), pl.ds(q * 32, 32)],
                    vbuf.at[b], lsem.at[b])

        def finish(it, b):
            cid = s + it * _NS

            @pl.when(cid < nchunks)
            def _():
                pltpu.make_async_copy(dst2.at[pl.ds(cid, 1)],
                                      didx.at[pl.ds(b, 1)], lsem.at[b]).wait()
                pltpu.make_async_copy(
                    v.at[pl.ds(cid * 128, 128), pl.ds(q * 32, 32)],
                    vbuf.at[b], lsem.at[b]).wait()
                pltpu.sync_copy(vbuf.at[b], acc.at[didx.at[b]], add=True)

        start_loads(0, 0)
        start_loads(1, 1)

        def outer(o, _):
            it = o * 2
            finish(it, 0)
            start_loads(it + 2, 0)
            finish(it + 1, 1)
            start_loads(it + 3, 1)

        lax.fori_loop(0, osteps, outer, None)
        plsc.subcore_barrier()
        pltpu.sync_copy(acc.at[pl.ds(s * rows, rows)],
                        acc_out.at[q, pl.ds(s * rows, rows)])
        plsc.subcore_barrier()


def _run_k4(dst2, v, n_nodes, e_total):
    mesh = plsc.VectorSubcoreMesh(core_axis_name="c", subcore_axis_name="s",
                                  num_cores=_NC, num_subcores=_NS)
    nchunks = e_total // 128
    rows = n_nodes // _NS
    zeros = jnp.zeros((rows, 32), jnp.float32)
    f = pl.kernel(
        functools.partial(_k4_body, nchunks, n_nodes),
        out_type=jax.ShapeDtypeStruct((4, n_nodes, 32), jnp.float32),
        mesh=mesh,
        scratch_types=[
            pltpu.VMEM((2, 128), jnp.int32),
            pltpu.VMEM((2, 128, 32), jnp.float32),
            pltpu.VMEM_SHARED((n_nodes, 32), jnp.float32),
            pltpu.SemaphoreType.DMA((2,)),
        ],
        compiler_params=pltpu.CompilerParams(use_tc_tiling_on_sc=False),
    )
    return f(dst2, v, zeros)


def _k5a_body(acc_ref, wpost_ref, bpost_ref, z_ref, st_ref):
    a = acc_ref[...]
    agg = jnp.concatenate(
        [a[qq, :, 16:32] / (a[qq, :, 0:16] + 1e-16) for qq in range(4)], axis=1)
    z = jnp.dot(agg, wpost_ref[...]) + bpost_ref[...]
    z_ref[...] = z

    @pl.when(pl.program_id(0) == 0)
    def _():
        st_ref[...] = jnp.zeros_like(st_ref)

    st_ref[0:1, :] += jnp.sum(z, axis=0, keepdims=True)
    st_ref[1:2, :] += jnp.sum(z * z, axis=0, keepdims=True)


def _run_k5a(acc, w_post, b_post, n_nodes, bn):
    return pl.pallas_call(
        _k5a_body,
        grid=(n_nodes // bn,),
        in_specs=[pl.BlockSpec((4, bn, 32), lambda i: (0, i, 0)),
                  pl.BlockSpec((64, 64), lambda i: (0, 0)),
                  pl.BlockSpec((1, 64), lambda i: (0, 0))],
        out_specs=[pl.BlockSpec((bn, 64), lambda i: (i, 0)),
                   pl.BlockSpec((2, 64), lambda i: (0, 0))],
        out_shape=[jax.ShapeDtypeStruct((n_nodes, 64), jnp.float32),
                   jax.ShapeDtypeStruct((2, 64), jnp.float32)],
    )(acc, w_post, b_post)


def _k5b_body(z_ref, scale_ref, shift_ref, out_ref):
    y = z_ref[...] * scale_ref[...] + shift_ref[...]
    out_ref[...] = jnp.where(y >= 0, y, 0.2 * y)


def _run_k5b(z, scale, shift, n_nodes, bn):
    return pl.pallas_call(
        _k5b_body,
        grid=(n_nodes // bn,),
        in_specs=[pl.BlockSpec((bn, 64), lambda i: (i, 0)),
                  pl.BlockSpec((1, 64), lambda i: (0, 0)),
                  pl.BlockSpec((1, 64), lambda i: (0, 0))],
        out_specs=pl.BlockSpec((bn, 64), lambda i: (i, 0)),
        out_shape=jax.ShapeDtypeStruct((n_nodes, 64), jnp.float32),
    )(z, scale, shift)


def kernel(edge_index, x, pos, normals, W_enc, b_enc, g_enc, be_enc, W_att,
           W_post, b_post, g_post, be_post):
    n = x.shape[0]
    e = edge_index.shape[1]
    assert e % 256 == 0 and n % _NS == 0
    be2 = _pick_block(e, [1280, 640, 512, 256, 128, 64, 32, 16, 8])
    be3 = _pick_block(e, [1280, 640, 512, 256, 128, 64, 32, 16, 8])
    bn = _pick_block(n, [1000, 512, 500, 256, 200, 128, 64, 32, 16, 8])

    src2 = edge_index[0].reshape(e // 128, 128).astype(jnp.int32)
    dst2 = edge_index[1].reshape(e // 128, 128).astype(jnp.int32)
    gtab = jnp.concatenate(
        [pos, normals, jnp.zeros((n, 10), jnp.float32)], axis=1)

    gsrc, gdst, xj = _run_k1(src2, dst2, gtab, x, e)

    rel, stats = _run_k2(gsrc, gdst, W_enc, b_enc.reshape(32, 1), e, be2)
    m = stats[:, 0] / e
    var = stats[:, 1] / e - m * m
    inv = g_enc / jnp.sqrt(var + 1e-6)
    scale = inv.reshape(1, 32)
    shift = (be_enc - m * inv).reshape(1, 32)

    v = _run_k3(rel, xj, W_enc, scale, shift, W_att, e, be3)

    acc = _run_k4(dst2, v, n, e)

    z, zst = _run_k5a(acc, W_post, b_post.reshape(1, 64), n, bn)
    m2 = zst[0] / n
    var2 = zst[1] / n - m2 * m2
    inv2 = g_post / jnp.sqrt(var2 + 1e-6)
    scale2 = inv2.reshape(1, 64)
    shift2 = (be_post - m2 * inv2).reshape(1, 64)

    return _run_k5b(z, scale2, shift2, n, bn)


# speedup vs baseline: 10.6468x; 1.0642x over previous
import functools

import jax
import jax.numpy as jnp
from jax import lax
from jax.experimental import pallas as pl
from jax.experimental.pallas import tpu as pltpu
from jax.experimental.pallas import tpu_sc as plsc

_NC = 2
_NS = 16
_NW = _NC * _NS


def _pick_block(n, cands):
    for c in cands:
        if n % c == 0:
            return c
    raise ValueError(f"no block size for {n}")


def _k1_body(nchunks, src2, dst2, gtab, xtab, gsrc, gdst, xj, sidx, didx,
             gs, gd, xb, isem, gsem, wsem):
    wid = lax.axis_index("s") * _NC + lax.axis_index("c")
    iters = -(-nchunks // _NW)
    osteps = -(-iters // 2)

    def start_idx(it, b):
        cid = wid + it * _NW

        @pl.when(cid < nchunks)
        def _():
            pltpu.async_copy(src2.at[pl.ds(2 * cid, 2)], sidx.at[b],
                             isem.at[b])
            pltpu.async_copy(dst2.at[pl.ds(2 * cid, 2)], didx.at[b],
                             isem.at[b])

    def wait_idx(it, b):
        cid = wid + it * _NW

        @pl.when(cid < nchunks)
        def _():
            pltpu.make_async_copy(src2.at[pl.ds(2 * cid, 2)], sidx.at[b],
                                  isem.at[b]).wait()
            pltpu.make_async_copy(dst2.at[pl.ds(2 * cid, 2)], didx.at[b],
                                  isem.at[b]).wait()

    def start_gathers(it, b):
        cid = wid + it * _NW

        @pl.when(cid < nchunks)
        def _():
            for h in range(2):
                pltpu.async_copy(gtab.at[sidx.at[b, h]],
                                 gs.at[b, pl.ds(h * 128, 128)], gsem.at[b])
                pltpu.async_copy(gtab.at[didx.at[b, h]],
                                 gd.at[b, pl.ds(h * 128, 128)], gsem.at[b])
                pltpu.async_copy(xtab.at[sidx.at[b, h]],
                                 xb.at[b, pl.ds(h * 128, 128)], gsem.at[b])

    def wait_gathers(it, b):
        cid = wid + it * _NW

        @pl.when(cid < nchunks)
        def _():
            for h in range(2):
                pltpu.make_async_copy(gtab.at[sidx.at[b, h]],
                                      gs.at[b, pl.ds(h * 128, 128)],
                                      gsem.at[b]).wait()
                pltpu.make_async_copy(gtab.at[didx.at[b, h]],
                                      gd.at[b, pl.ds(h * 128, 128)],
                                      gsem.at[b]).wait()
                pltpu.make_async_copy(xtab.at[sidx.at[b, h]],
                                      xb.at[b, pl.ds(h * 128, 128)],
                                      gsem.at[b]).wait()

    def writes(it, b):
        cid = wid + it * _NW

        @pl.when(cid < nchunks)
        def _():
            base = cid * 256
            w1 = pltpu.async_copy(gs.at[b], gsrc.at[pl.ds(base, 256), :], wsem)
            w2 = pltpu.async_copy(gd.at[b], gdst.at[pl.ds(base, 256), :], wsem)
            w3 = pltpu.async_copy(xb.at[b], xj.at[pl.ds(base, 256), :], wsem)
            w1.wait()
            w2.wait()
            w3.wait()

    def step(it, b):
        wait_idx(it + 1, 1 - b)
        start_gathers(it + 1, 1 - b)
        wait_gathers(it, b)
        start_idx(it + 2, b)
        writes(it, b)

    start_idx(0, 0)
    wait_idx(0, 0)
    start_gathers(0, 0)
    start_idx(1, 1)

    def outer(o, _):
        it = o * 2
        step(it, 0)
        step(it + 1, 1)

    lax.fori_loop(0, osteps, outer, None)


def _run_k1(src2, dst2, gtab, xtab, e_total):
    mesh = plsc.VectorSubcoreMesh(core_axis_name="c", subcore_axis_name="s",
                                  num_cores=_NC, num_subcores=_NS)
    nchunks = e_total // 256
    f = pl.kernel(
        functools.partial(_k1_body, nchunks),
        out_type=[jax.ShapeDtypeStruct((e_total, 16), jnp.float32),
                  jax.ShapeDtypeStruct((e_total, 16), jnp.float32),
                  jax.ShapeDtypeStruct((e_total, 32), jnp.float32)],
        mesh=mesh,
        scratch_types=[
            pltpu.VMEM((2, 2, 128), jnp.int32),
            pltpu.VMEM((2, 2, 128), jnp.int32),
            pltpu.VMEM((2, 256, 16), jnp.float32),
            pltpu.VMEM((2, 256, 16), jnp.float32),
            pltpu.VMEM((2, 256, 32), jnp.float32),
            pltpu.SemaphoreType.DMA((2,)),
            pltpu.SemaphoreType.DMA((2,)),
            pltpu.SemaphoreType.DMA,
        ],
        compiler_params=pltpu.CompilerParams(use_tc_tiling_on_sc=False),
    )
    return f(src2, dst2, gtab, xtab)


def _rel_t(gs, gd):
    eye = jax.lax.broadcasted_iota(jnp.int32, (16, 16), 0) == \
        jax.lax.broadcasted_iota(jnp.int32, (16, 16), 1)
    eye = eye.astype(jnp.float32)
    gst = lax.dot_general(eye, gs, (((1,), (1,)), ((), ())))
    gdt = lax.dot_general(eye, gd, (((1,), (1,)), ((), ())))
    d0 = gst[0:1] - gdt[0:1]
    d1 = gst[1:2] - gdt[1:2]
    d2 = gst[2:3] - gdt[2:3]
    a0, a1, a2 = gdt[3:4], gdt[4:5], gdt[5:6]
    b0, b1, b2 = gst[3:4], gst[4:5], gst[5:6]
    dist = jnp.sqrt(d0 * d0 + d1 * d1 + d2 * d2)
    li = jnp.sqrt(a0 * a0 + a1 * a1 + a2 * a2)
    lj = jnp.sqrt(b0 * b0 + b1 * b1 + b2 * b2)
    f0 = dist
    f1 = (d0 * a0 + d1 * a1 + d2 * a2) / (dist * li + 1e-10)
    f2 = (d0 * b0 + d1 * b1 + d2 * b2) / (dist * lj + 1e-10)
    f3 = (a0 * b0 + a1 * b1 + a2 * b2) / (li * lj + 1e-10)
    uq0, uq1, uq2 = d1 * a2 - d2 * a1, d2 * a0 - d0 * a2, d0 * a1 - d1 * a0
    vq0, vq1, vq2 = uq1 * a2 - uq2 * a1, uq2 * a0 - uq0 * a2, uq0 * a1 - uq1 * a0
    uk0, uk1, uk2 = d1 * b2 - d2 * b1, d2 * b0 - d0 * b2, d0 * b1 - d1 * b0
    vk0, vk1, vk2 = uk1 * b2 - uk2 * b1, uk2 * b0 - uk0 * b2, uk0 * b1 - uk1 * b0
    luq = jnp.sqrt(uq0 * uq0 + uq1 * uq1 + uq2 * uq2)
    lvq = jnp.sqrt(vq0 * vq0 + vq1 * vq1 + vq2 * vq2)
    luk = jnp.sqrt(uk0 * uk0 + uk1 * uk1 + uk2 * uk2)
    lvk = jnp.sqrt(vk0 * vk0 + vk1 * vk1 + vk2 * vk2)
    f4 = (uq0 * uk0 + uq1 * uk1 + uq2 * uk2) / (luq * luk + 1e-10)
    f5 = (vq0 * vk0 + vq1 * vk1 + vq2 * vk2) / (lvq * lvk + 1e-10)
    f6 = (uq0 * vk0 + uq1 * vk1 + uq2 * vk2) / (luq * lvk + 1e-10)
    f7 = (vq0 * uk0 + vq1 * uk1 + vq2 * uk2) / (lvq * luk + 1e-10)
    return jnp.concatenate([f0, f1, f2, f3, f4, f5, f6, f7], axis=0)


def _k2_body(gs_ref, gd_ref, wenc_ref, benc_ref, rel_ref, out_ref):
    relT = _rel_t(gs_ref[...], gd_ref[...])
    rel_ref[...] = relT
    hT = lax.dot_general(wenc_ref[...], relT, (((0,), (0,)), ((), ())))
    hT = hT + benc_ref[...]

    @pl.when(pl.program_id(0) == 0)
    def _():
        out_ref[...] = jnp.zeros_like(out_ref)

    out_ref[:, 0:1] += jnp.sum(hT, axis=1, keepdims=True)
    out_ref[:, 1:2] += jnp.sum(hT * hT, axis=1, keepdims=True)


def _run_k2(gsrc, gdst, w_enc, b_enc, e_total, be):
    return pl.pallas_call(
        _k2_body,
        grid=(e_total // be,),
        in_specs=[pl.BlockSpec((be, 16), lambda i: (i, 0)),
                  pl.BlockSpec((be, 16), lambda i: (i, 0)),
                  pl.BlockSpec((8, 32), lambda i: (0, 0)),
                  pl.BlockSpec((32, 1), lambda i: (0, 0))],
        out_specs=[pl.BlockSpec((8, be), lambda i: (0, i)),
                   pl.BlockSpec((32, 2), lambda i: (0, 0))],
        out_shape=[jax.ShapeDtypeStruct((8, e_total), jnp.float32),
                   jax.ShapeDtypeStruct((32, 2), jnp.float32)],
    )(gsrc, gdst, w_enc, b_enc)


def _k3_body(rel_ref, xj_ref, wenc_ref, scale_ref, shift_ref, watt_ref, v_ref):
    h = lax.dot_general(rel_ref[...], wenc_ref[...],
                        (((0,), (0,)), ((), ())))
    enc = h * scale_ref[...] + shift_ref[...]
    enc = jnp.where(enc >= 0, enc, 0.2 * enc)
    local = jnp.concatenate([xj_ref[...], enc], axis=1)
    att = jnp.dot(local, watt_ref[...])
    att = jnp.minimum(jnp.maximum(att, -60.0), 60.0)
    e = jnp.exp(att)
    w = e * local
    row = jax.lax.broadcasted_iota(jnp.int32, (64, 128), 0)
    col = jax.lax.broadcasted_iota(jnp.int32, (64, 128), 1)
    qq, rr = col // 32, col % 32
    pe = ((rr < 16) & (row == qq * 16 + rr)).astype(jnp.float32)
    pw = ((rr >= 16) & (row == qq * 16 + rr - 16)).astype(jnp.float32)
    v_ref[...] = jnp.dot(e, pe) + jnp.dot(w, pw)


def _run_k3(rel, xj, w_enc, scale, shift, w_att, e_half, be, ob):
    return pl.pallas_call(
        _k3_body,
        grid=(e_half // be,),
        in_specs=[pl.BlockSpec((8, be), lambda i: (0, i + ob)),
                  pl.BlockSpec((be, 32), lambda i: (i + ob, 0)),
                  pl.BlockSpec((8, 32), lambda i: (0, 0)),
                  pl.BlockSpec((1, 32), lambda i: (0, 0)),
                  pl.BlockSpec((1, 32), lambda i: (0, 0)),
                  pl.BlockSpec((64, 64), lambda i: (0, 0))],
        out_specs=pl.BlockSpec((be, 128), lambda i: (i, 0)),
        out_shape=jax.ShapeDtypeStruct((e_half, 128), jnp.float32),
    )(rel, xj, w_enc, scale, shift, w_att)


def _k4_body(nchunks, n_nodes, coff, chain, dst2, v, init, acc_out,
             didx, vbuf, acc, lsem):
    c = lax.axis_index("c")
    s = lax.axis_index("s")
    rows = n_nodes // _NS
    iters = -(-nchunks // _NS)
    osteps = -(-iters // 2)

    for p in range(2):
        q = c + 2 * p
        if chain:
            pltpu.sync_copy(init.at[q, pl.ds(s * rows, rows)],
                            acc.at[pl.ds(s * rows, rows)])
        else:
            pltpu.sync_copy(init, acc.at[pl.ds(s * rows, rows)])
        plsc.subcore_barrier()

        def start_loads(it, b):
            cid = s + it * _NS

            @pl.when(cid < nchunks)
            def _():
                pltpu.async_copy(dst2.at[pl.ds(coff + cid, 1)],
                                 didx.at[pl.ds(b, 1)], lsem.at[b])
                pltpu.async_copy(
                    v.at[pl.ds(cid * 128, 128), pl.ds(q * 32, 32)],
                    vbuf.at[b], lsem.at[b])

        def finish(it, b):
            cid = s + it * _NS

            @pl.when(cid < nchunks)
            def _():
                pltpu.make_async_copy(dst2.at[pl.ds(coff + cid, 1)],
                                      didx.at[pl.ds(b, 1)], lsem.at[b]).wait()
                pltpu.make_async_copy(
                    v.at[pl.ds(cid * 128, 128), pl.ds(q * 32, 32)],
                    vbuf.at[b], lsem.at[b]).wait()
                pltpu.sync_copy(vbuf.at[b], acc.at[didx.at[b]], add=True)

        start_loads(0, 0)
        start_loads(1, 1)

        def outer(o, _):
            it = o * 2
            finish(it, 0)
            start_loads(it + 2, 0)
            finish(it + 1, 1)
            start_loads(it + 3, 1)

        lax.fori_loop(0, osteps, outer, None)
        plsc.subcore_barrier()
        pltpu.sync_copy(acc.at[pl.ds(s * rows, rows)],
                        acc_out.at[q, pl.ds(s * rows, rows)])
        plsc.subcore_barrier()


def _run_k4(dst2, v, init, n_nodes, e_half, coff, chain):
    mesh = plsc.VectorSubcoreMesh(core_axis_name="c", subcore_axis_name="s",
                                  num_cores=_NC, num_subcores=_NS)
    nchunks = e_half // 128
    f = pl.kernel(
        functools.partial(_k4_body, nchunks, n_nodes, coff, chain),
        out_type=jax.ShapeDtypeStruct((4, n_nodes, 32), jnp.float32),
        mesh=mesh,
        scratch_types=[
            pltpu.VMEM((2, 128), jnp.int32),
            pltpu.VMEM((2, 128, 32), jnp.float32),
            pltpu.VMEM_SHARED((n_nodes, 32), jnp.float32),
            pltpu.SemaphoreType.DMA((2,)),
        ],
        compiler_params=pltpu.CompilerParams(use_tc_tiling_on_sc=False),
    )
    return f(dst2, v, init)


def _k5a_body(acc_ref, wpost_ref, bpost_ref, z_ref, st_ref):
    a = acc_ref[...]
    agg = jnp.concatenate(
        [a[qq, :, 16:32] / (a[qq, :, 0:16] + 1e-16) for qq in range(4)], axis=1)
    z = jnp.dot(agg, wpost_ref[...]) + bpost_ref[...]
    z_ref[...] = z

    @pl.when(pl.program_id(0) == 0)
    def _():
        st_ref[...] = jnp.zeros_like(st_ref)

    st_ref[0:1, :] += jnp.sum(z, axis=0, keepdims=True)
    st_ref[1:2, :] += jnp.sum(z * z, axis=0, keepdims=True)


def _run_k5a(acc, w_post, b_post, n_nodes, bn):
    return pl.pallas_call(
        _k5a_body,
        grid=(n_nodes // bn,),
        in_specs=[pl.BlockSpec((4, bn, 32), lambda i: (0, i, 0)),
                  pl.BlockSpec((64, 64), lambda i: (0, 0)),
                  pl.BlockSpec((1, 64), lambda i: (0, 0))],
        out_specs=[pl.BlockSpec((bn, 64), lambda i: (i, 0)),
                   pl.BlockSpec((2, 64), lambda i: (0, 0))],
        out_shape=[jax.ShapeDtypeStruct((n_nodes, 64), jnp.float32),
                   jax.ShapeDtypeStruct((2, 64), jnp.float32)],
    )(acc, w_post, b_post)


def _k5b_body(z_ref, scale_ref, shift_ref, out_ref):
    y = z_ref[...] * scale_ref[...] + shift_ref[...]
    out_ref[...] = jnp.where(y >= 0, y, 0.2 * y)


def _run_k5b(z, scale, shift, n_nodes, bn):
    return pl.pallas_call(
        _k5b_body,
        grid=(n_nodes // bn,),
        in_specs=[pl.BlockSpec((bn, 64), lambda i: (i, 0)),
                  pl.BlockSpec((1, 64), lambda i: (0, 0)),
                  pl.BlockSpec((1, 64), lambda i: (0, 0))],
        out_specs=pl.BlockSpec((bn, 64), lambda i: (i, 0)),
        out_shape=jax.ShapeDtypeStruct((n_nodes, 64), jnp.float32),
    )(z, scale, shift)


def kernel(edge_index, x, pos, normals, W_enc, b_enc, g_enc, be_enc, W_att,
           W_post, b_post, g_post, be_post):
    n = x.shape[0]
    e = edge_index.shape[1]
    assert e % 256 == 0 and n % _NS == 0
    be2 = _pick_block(e, [1280, 640, 512, 256, 128, 64, 32, 16, 8])
    be3 = _pick_block(e, [1280, 640, 512, 256, 128, 64, 32, 16, 8])
    bn = _pick_block(n, [1000, 512, 500, 256, 200, 128, 64, 32, 16, 8])

    src2 = edge_index[0].reshape(e // 128, 128).astype(jnp.int32)
    dst2 = edge_index[1].reshape(e // 128, 128).astype(jnp.int32)
    gtab = jnp.concatenate(
        [pos, normals, jnp.zeros((n, 10), jnp.float32)], axis=1)

    gsrc, gdst, xj = _run_k1(src2, dst2, gtab, x, e)

    rel, stats = _run_k2(gsrc, gdst, W_enc, b_enc.reshape(32, 1), e, be2)
    m = stats[:, 0] / e
    var = stats[:, 1] / e - m * m
    inv = g_enc / jnp.sqrt(var + 1e-6)
    scale = inv.reshape(1, 32)
    shift = (be_enc - m * inv).reshape(1, 32)

    nb = e // be3
    nb0 = nb // 2
    eh0 = nb0 * be3
    eh1 = e - eh0
    v0 = _run_k3(rel, xj, W_enc, scale, shift, W_att, eh0, be3, 0)
    v1 = _run_k3(rel, xj, W_enc, scale, shift, W_att, eh1, be3, nb0)
    zeros = jnp.zeros((n // _NS, 32), jnp.float32)
    acc0 = _run_k4(dst2, v0, zeros, n, eh0, 0, False)
    acc = _run_k4(dst2, v1, acc0, n, eh1, eh0 // 128, True)

    z, zst = _run_k5a(acc, W_post, b_post.reshape(1, 64), n, bn)
    m2 = zst[0] / n
    var2 = zst[1] / n - m2 * m2
    inv2 = g_post / jnp.sqrt(var2 + 1e-6)
    scale2 = inv2.reshape(1, 64)
    shift2 = (be_post - m2 * inv2).reshape(1, 64)

    return _run_k5b(z, scale2, shift2, n, bn)
